# baseline (speedup 1.0000x reference)
import functools

import jax
import jax.numpy as jnp
from jax import lax
from jax.experimental import pallas as pl
from jax.experimental.pallas import tpu as pltpu
from jax.experimental.pallas import tpu_sc as plsc

D = 128
W_AUG = 144
NC, NS, LANES = 2, 16, 16
NW = NC * NS
BATCH = 128
GROUP = 1
FRAC_CORE0 = 0.85
ICH = 4


def _agg_body(n_nodes, n_pad, r0pt, r1pt, table, sdidx, out,
              idx2, dummy_idx, rb0, rb1, sem0, sem1, acc):
    cid = lax.axis_index("c")
    sid = lax.axis_index("s")
    bufs = (rb0, rb1)
    sems = (sem0, sem1)
    ng = jnp.where(cid == 0, r0pt, r1pt)
    base = jnp.where(cid == 0, sid * r0pt, NS * r0pt + sid * r1pt)

    def zr(r, carry):
        for buf in bufs:
            for cc in range(W_AUG // LANES):
                buf[r, pl.ds(cc * LANES, LANES)] = jnp.zeros(
                    (LANES,), jnp.float32)
        return carry
    lax.fori_loop(0, BATCH, zr, 0)
    rpw = n_pad // NS
    for k in range(rpw // BATCH):
        pltpu.sync_copy(rb0, acc.at[pl.ds(sid * rpw + k * BATCH, BATCH)])
    zrem = rpw % BATCH
    if zrem:
        pltpu.sync_copy(rb0.at[pl.ds(0, zrem)],
                        acc.at[pl.ds(sid * rpw + (rpw - zrem), zrem)])
    for cc in range(BATCH // LANES):
        dummy_idx[0, pl.ds(cc * LANES, LANES)] = jnp.full(
            (LANES,), n_nodes, jnp.int32)
    plsc.subcore_barrier()

    pltpu.sync_copy(sdidx.at[pl.ds(base, 1)], idx2.at[pl.ds(0, 1)])
    pltpu.async_copy(rb1, acc.at[dummy_idx.at[0]], sem1, add=True)
    pltpu.async_copy(table.at[idx2.at[0, 0]], rb0, sem0)

    def half(g, p):
        q = 1 - p
        pltpu.make_async_copy(table.at[idx2.at[p, 0]], bufs[p],
                              sems[p]).wait()
        pltpu.make_async_copy(bufs[q], acc.at[dummy_idx.at[0]],
                              sems[q]).wait()
        r = base + lax.rem(g + 1, ng)
        pltpu.sync_copy(sdidx.at[pl.ds(r, 1)], idx2.at[pl.ds(q, 1)])
        pltpu.async_copy(table.at[idx2.at[q, 0]], bufs[q], sems[q])
        pltpu.async_copy(bufs[p], acc.at[idx2.at[p, 1]], sems[p],
                         add=True)

    def body(t, carry):
        half(2 * t, 0)
        half(2 * t + 1, 1)
        return carry
    lax.fori_loop(0, ng // 2, body, 0)

    pltpu.make_async_copy(table.at[idx2.at[0, 0]], rb0, sem0).wait()
    pltpu.make_async_copy(rb1, acc.at[dummy_idx.at[0]], sem1).wait()
    plsc.subcore_barrier()

    pltpu.sync_copy(acc.at[pl.ds(sid * rpw, rpw)],
                    out.at[cid, pl.ds(sid * rpw, rpw)])


def _make_agg(n_nodes, n_pad, r0pt, r1pt):
    mesh = plsc.VectorSubcoreMesh(core_axis_name="c", subcore_axis_name="s")
    return pl.kernel(
        functools.partial(_agg_body, n_nodes, n_pad, r0pt, r1pt),
        out_type=jax.ShapeDtypeStruct((NC, n_pad, W_AUG), jnp.float32),
        mesh=mesh,
        compiler_params=pltpu.CompilerParams(use_tc_tiling_on_sc=False),
        scratch_types=[
            pltpu.VMEM((2, 2, BATCH), jnp.int32),
            pltpu.VMEM((1, BATCH), jnp.int32),
            pltpu.VMEM((BATCH, W_AUG), jnp.float32),
            pltpu.VMEM((BATCH, W_AUG), jnp.float32),
            pltpu.SemaphoreType.DMA,
            pltpu.SemaphoreType.DMA,
            pltpu.VMEM_SHARED((n_pad, W_AUG), jnp.float32),
        ],
    )


def _tc_layer(p0, p1, root, wl, bl, wr, make_next):
    n = root.shape[0]
    rb = 400
    grid = (n // rb,)
    rw = root.shape[1]

    def body(p0_ref, p1_ref, x_ref, wl_ref, bl_ref, wr_ref, o1_ref, *rest):
        s = p0_ref[...] + p1_ref[...]
        cnt = s[:, D:D + 1]
        mean = s[:, :D] / jnp.maximum(cnt, 1.0)
        xr = x_ref[...][:, :D]
        h1 = (jnp.dot(mean, wl_ref[...], preferred_element_type=jnp.float32)
              + bl_ref[...]
              + jnp.dot(xr, wr_ref[...], preferred_element_type=jnp.float32))
        o1_ref[...] = h1
        if make_next:
            h = jnp.maximum(h1, 0.0)
            aug = jnp.pad(h, ((0, 0), (0, W_AUG - D)))
            col = lax.broadcasted_iota(jnp.int32, (rb, W_AUG), 1)
            rest[0][...] = jnp.where(col == D, 1.0, aug)

    out_shape = [jax.ShapeDtypeStruct((n, D), jnp.float32)]
    if make_next:
        out_shape.append(jax.ShapeDtypeStruct((n, W_AUG), jnp.float32))
    outs = pl.pallas_call(
        body,
        grid=grid,
        in_specs=[
            pl.BlockSpec((rb, W_AUG), lambda i: (i, 0)),
            pl.BlockSpec((rb, W_AUG), lambda i: (i, 0)),
            pl.BlockSpec((rb, rw), lambda i: (i, 0)),
            pl.BlockSpec((D, D), lambda i: (0, 0)),
            pl.BlockSpec((1, D), lambda i: (0, 0)),
            pl.BlockSpec((D, D), lambda i: (0, 0)),
        ],
        out_specs=[pl.BlockSpec((rb, D), lambda i: (i, 0))]
        + ([pl.BlockSpec((rb, W_AUG), lambda i: (i, 0))] if make_next else []),
        out_shape=out_shape,
    )(p0, p1, root, wl, bl.reshape(1, D), wr)
    return outs


def kernel(x, adj_t, Wl0, bl0, Wr0, Wl1, bl1, Wr1):
    n = x.shape[0]
    src = adj_t[0].astype(jnp.int32)
    dst = adj_t[1].astype(jnp.int32)
    e = src.shape[0]

    unit = NW * BATCH * GROUP * 2
    e_pad = ((e + unit - 1) // unit) * unit
    pad = e_pad - e
    src_p = jnp.concatenate([src, jnp.zeros((pad,), jnp.int32)])
    dst_p = jnp.concatenate([dst, jnp.full((pad,), n, jnp.int32)])
    src2d = src_p.reshape(e_pad // BATCH, BATCH)
    dst2d = dst_p.reshape(e_pad // BATCH, BATCH)
    sdidx = jnp.stack([src2d, dst2d], axis=1)

    nrows = e_pad // BATCH
    r0pt = max(2 * ICH, int(round(FRAC_CORE0 * nrows / NS / (2 * ICH))) * 2 * ICH)
    r1pt = nrows // NS - r0pt

    zunit = NS * BATCH
    n_pad = ((n + 1 + zunit - 1) // zunit) * zunit

    agg = _make_agg(n, n_pad, r0pt, r1pt)

    x_aug = jnp.concatenate(
        [x, jnp.ones((n, 1), jnp.float32), jnp.zeros((n, W_AUG - D - 1), jnp.float32)],
        axis=1)

    p = agg(x_aug, sdidx)
    h1, h_aug = _tc_layer(p[0], p[1], x, Wl0, bl0, Wr0, make_next=True)
    p2 = agg(h_aug, sdidx)
    h2 = _tc_layer(p2[0], p2[1], h_aug, Wl1, bl1, Wr1, make_next=False)[0]
    return (h1, h2)


# speedup vs baseline: 1.0458x; 1.0458x over previous
import functools

import jax
import jax.numpy as jnp
from jax import lax
from jax.experimental import pallas as pl
from jax.experimental.pallas import tpu as pltpu
from jax.experimental.pallas import tpu_sc as plsc

D = 128
W_AUG = 144
NC, NS, LANES = 2, 16, 16
NW = NC * NS
BATCH = 128
GROUP = 1
FRAC_CORE0 = 0.85
ICH = 4


def _agg_body(n_nodes, n_pad, r0pt, r1pt, table, src2d, dst2d, out,
              idx_s, idx_d, r00, r10, s00, s10, acc):
    cid = lax.axis_index("c")
    sid = lax.axis_index("s")
    bufs = ((r00,), (r10,))
    sems = ((s00,), (s10,))
    ngroups = jnp.where(cid == 0, r0pt, r1pt)
    row_base = jnp.where(cid == 0, sid * r0pt, NS * r0pt + sid * r1pt)

    def zr(r, carry):
        for s in range(2):
            for b in range(GROUP):
                for cc in range(W_AUG // LANES):
                    bufs[s][b][r, pl.ds(cc * LANES, LANES)] = jnp.zeros(
                        (LANES,), jnp.float32)
        return carry
    lax.fori_loop(0, BATCH, zr, 0)
    zchunks = n_pad // NS // BATCH
    for k in range(zchunks):
        pltpu.sync_copy(r00, acc.at[pl.ds((sid * zchunks + k) * BATCH, BATCH)])
    for b in range(GROUP):
        for cc in range(BATCH // LANES):
            idx_d[1, b, pl.ds(cc * LANES, LANES)] = jnp.full(
                (LANES,), n_nodes, jnp.int32)
    plsc.subcore_barrier()

    for b in range(GROUP):
        pltpu.async_copy(bufs[1][b], acc.at[idx_d.at[1, b]], sems[1][b],
                         add=True)
    pltpu.sync_copy(src2d.at[pl.ds(row_base, GROUP)], idx_s.at[0])
    pltpu.sync_copy(dst2d.at[pl.ds(row_base, GROUP)], idx_d.at[0])
    for b in range(GROUP):
        pltpu.async_copy(table.at[idx_s.at[0, b]], bufs[0][b], sems[0][b])

    def half(g, p):
        q = 1 - p
        for b in range(GROUP):
            pltpu.make_async_copy(table.at[idx_s.at[p, b]], bufs[p][b],
                                  sems[p][b]).wait()
        for b in range(GROUP):
            pltpu.make_async_copy(bufs[q][b], acc.at[idx_d.at[q, b]],
                                  sems[q][b]).wait()
        roff = row_base + lax.rem(g + 1, ngroups) * GROUP
        pltpu.sync_copy(src2d.at[pl.ds(roff, GROUP)], idx_s.at[q])
        pltpu.sync_copy(dst2d.at[pl.ds(roff, GROUP)], idx_d.at[q])
        for b in range(GROUP):
            pltpu.async_copy(table.at[idx_s.at[q, b]], bufs[q][b], sems[q][b])
        for b in range(GROUP):
            pltpu.async_copy(bufs[p][b], acc.at[idx_d.at[p, b]], sems[p][b],
                             add=True)

    def body(t, carry):
        half(2 * t, 0)
        half(2 * t + 1, 1)
        return carry
    lax.fori_loop(0, ngroups // 2, body, 0)

    for b in range(GROUP):
        pltpu.make_async_copy(table.at[idx_s.at[0, b]], bufs[0][b],
                              sems[0][b]).wait()
    for b in range(GROUP):
        pltpu.make_async_copy(bufs[1][b], acc.at[idx_d.at[1, b]],
                              sems[1][b]).wait()
    plsc.subcore_barrier()

    rpw = n_pad // NS
    pltpu.sync_copy(acc.at[pl.ds(sid * rpw, rpw)],
                    out.at[cid, pl.ds(sid * rpw, rpw)])


def _make_agg(n_nodes, n_pad, r0pt, r1pt):
    mesh = plsc.VectorSubcoreMesh(core_axis_name="c", subcore_axis_name="s")
    return pl.kernel(
        functools.partial(_agg_body, n_nodes, n_pad, r0pt, r1pt),
        out_type=jax.ShapeDtypeStruct((NC, n_pad, W_AUG), jnp.float32),
        mesh=mesh,
        compiler_params=pltpu.CompilerParams(use_tc_tiling_on_sc=False),
        scratch_types=[
            pltpu.VMEM((2, GROUP, BATCH), jnp.int32),
            pltpu.VMEM((2, GROUP, BATCH), jnp.int32),
            pltpu.VMEM((BATCH, W_AUG), jnp.float32),
            pltpu.VMEM((BATCH, W_AUG), jnp.float32),
            pltpu.SemaphoreType.DMA,
            pltpu.SemaphoreType.DMA,
            pltpu.VMEM_SHARED((n_pad, W_AUG), jnp.float32),
        ],
    )


def _tc_layer(p0, p1, root, wl, bl, wr, make_next):
    n = root.shape[0]
    rb = 400
    grid = (n // rb,)
    rw = root.shape[1]

    def body(p0_ref, p1_ref, x_ref, wl_ref, bl_ref, wr_ref, o1_ref, *rest):
        s = p0_ref[...] + p1_ref[...]
        cnt = s[:, D:D + 1]
        mean = s[:, :D] / jnp.maximum(cnt, 1.0)
        xr = x_ref[...][:, :D]
        h1 = (jnp.dot(mean, wl_ref[...], preferred_element_type=jnp.float32)
              + bl_ref[...]
              + jnp.dot(xr, wr_ref[...], preferred_element_type=jnp.float32))
        o1_ref[...] = h1
        if make_next:
            h = jnp.maximum(h1, 0.0)
            aug = jnp.pad(h, ((0, 0), (0, W_AUG - D)))
            col = lax.broadcasted_iota(jnp.int32, (rb, W_AUG), 1)
            rest[0][...] = jnp.where(col == D, 1.0, aug)

    out_shape = [jax.ShapeDtypeStruct((n, D), jnp.float32)]
    if make_next:
        out_shape.append(jax.ShapeDtypeStruct((n, W_AUG), jnp.float32))
    outs = pl.pallas_call(
        body,
        grid=grid,
        in_specs=[
            pl.BlockSpec((rb, W_AUG), lambda i: (i, 0)),
            pl.BlockSpec((rb, W_AUG), lambda i: (i, 0)),
            pl.BlockSpec((rb, rw), lambda i: (i, 0)),
            pl.BlockSpec((D, D), lambda i: (0, 0)),
            pl.BlockSpec((1, D), lambda i: (0, 0)),
            pl.BlockSpec((D, D), lambda i: (0, 0)),
        ],
        out_specs=[pl.BlockSpec((rb, D), lambda i: (i, 0))]
        + ([pl.BlockSpec((rb, W_AUG), lambda i: (i, 0))] if make_next else []),
        out_shape=out_shape,
    )(p0, p1, root, wl, bl.reshape(1, D), wr)
    return outs


def kernel(x, adj_t, Wl0, bl0, Wr0, Wl1, bl1, Wr1):
    n = x.shape[0]
    src = adj_t[0].astype(jnp.int32)
    dst = adj_t[1].astype(jnp.int32)
    e = src.shape[0]

    unit = NW * BATCH * GROUP * 2
    e_pad = ((e + unit - 1) // unit) * unit
    pad = e_pad - e
    src_p = jnp.concatenate([src, jnp.zeros((pad,), jnp.int32)])
    dst_p = jnp.concatenate([dst, jnp.full((pad,), n, jnp.int32)])
    src2d = src_p.reshape(e_pad // BATCH, BATCH)
    dst2d = dst_p.reshape(e_pad // BATCH, BATCH)

    nrows = e_pad // BATCH
    r0pt = max(2, int(round(FRAC_CORE0 * nrows / NS / 2)) * 2)
    r1pt = nrows // NS - r0pt

    zunit = NS * BATCH
    n_pad = ((n + 1 + zunit - 1) // zunit) * zunit

    agg = _make_agg(n, n_pad, r0pt, r1pt)

    x_aug = jnp.concatenate(
        [x, jnp.ones((n, 1), jnp.float32), jnp.zeros((n, W_AUG - D - 1), jnp.float32)],
        axis=1)

    p = agg(x_aug, src2d, dst2d)
    h1, h_aug = _tc_layer(p[0], p[1], x, Wl0, bl0, Wr0, make_next=True)
    p2 = agg(h_aug, src2d, dst2d)
    h2 = _tc_layer(p2[0], p2[1], h_aug, Wl1, bl1, Wr1, make_next=False)[0]
    return (h1, h2)
